# baseline (speedup 1.0000x reference)
import functools

import jax
import jax.numpy as jnp
from jax import lax
from jax.experimental import pallas as pl
from jax.experimental.pallas import tpu as pltpu
from jax.experimental.pallas import tpu_sc as plsc

CLS = 19
_HB = 512
_SR = 8
_W = 512
_NW = 32
_BINS = 32
_NBUF = 2
_CHR = 16


def _focal_body(nh_blocks, x_ref, t_ref, out_ref, acc_ref):
    step = pl.program_id(0) * pl.num_programs(1) + pl.program_id(1)

    @pl.when(step == 0)
    def _init():
        acc_ref[...] = jnp.zeros_like(acc_ref)

    for r in range(_HB // _SR):
        rows = pl.ds(r * _SR, _SR)
        t = t_ref[0, rows]
        m = x_ref[0, 0, rows]
        for k in range(1, CLS):
            m = jnp.maximum(m, x_ref[0, k, rows])
        s = jnp.zeros((_SR, _W), jnp.float32)
        xt = jnp.zeros((_SR, _W), jnp.float32)
        for k in range(CLS):
            xk = x_ref[0, k, rows]
            s = s + jnp.exp(xk - m)
            xt = xt + jnp.where(t == k, xk, 0.0)
        log_p = xt - m - jnp.log(s)
        p = jnp.exp(log_p)
        om = 1.0 - p
        f = om * om * log_p
        for k in range(CLS):
            acc_ref[k] += jnp.where(t == k, f, 0.0)

    @pl.when(step == nh_blocks - 1)
    def _fin():
        out_ref[...] = jnp.sum(acc_ref[...], axis=(1, 2)).reshape(CLS, 1)


def _focal_sums(x, t32):
    n, c, h, w = x.shape
    grid = (n, h // _HB)
    return pl.pallas_call(
        functools.partial(_focal_body, grid[0] * grid[1]),
        grid=grid,
        in_specs=[
            pl.BlockSpec((1, CLS, _HB, _W), lambda i, j: (i, 0, j, 0)),
            pl.BlockSpec((1, _HB, _W), lambda i, j: (i, j, 0)),
        ],
        out_specs=pl.BlockSpec((CLS, 1), lambda i, j: (0, 0)),
        out_shape=jax.ShapeDtypeStruct((CLS, 1), jnp.float32),
        scratch_shapes=[pltpu.VMEM((CLS, _SR, _W), jnp.float32)],
    )(x, t32)


def _hist_body(rows_per_w, t_hbm, out_hbm, bufs, bins, sems):
    cid = lax.axis_index("c")
    sid = lax.axis_index("s")
    wid = sid * 2 + cid
    row0 = wid * rows_per_w
    nch = rows_per_w // _CHR

    zero = jnp.zeros((16,), jnp.float32)
    for b in range(4):
        for i in range(_BINS // 16):
            bins[b, pl.ds(i * 16, 16)] = zero
    ones = jnp.ones((16,), jnp.float32)

    def start(ci, slot):
        pltpu.async_copy(
            t_hbm.at[pl.ds(row0 + ci * _CHR, _CHR)], bufs.at[slot],
            sems.at[slot])

    for b in range(_NBUF):
        start(b, b)

    def process(ci, carry):
        slot = lax.rem(ci, _NBUF)
        pltpu.make_async_copy(
            t_hbm.at[pl.ds(row0 + ci * _CHR, _CHR)], bufs.at[slot],
            sems.at[slot]).wait()
        for rr in range(_CHR):
            for cc in range(_W // (16 * 4)):
                for u in range(4):
                    tv = bufs[slot, rr, pl.ds(cc * 64 + u * 16, 16)]
                    plsc.addupdate_scatter(bins.at[u], [tv], ones)

        @pl.when(ci + _NBUF < nch)
        def _next():
            start(ci + _NBUF, slot)

        return carry

    lax.fori_loop(0, nch, process, 0)
    for b in range(4):
        pltpu.sync_copy(bins.at[b], out_hbm.at[4 * wid + b])


def _class_hist(t2d):
    rows = t2d.shape[0]
    rows_per_w = rows // _NW
    mesh = plsc.VectorSubcoreMesh(core_axis_name="c", subcore_axis_name="s",
                                  num_cores=2, num_subcores=16)
    run = functools.partial(
        pl.kernel,
        out_type=jax.ShapeDtypeStruct((4 * _NW, _BINS), jnp.float32),
        mesh=mesh,
        scratch_types=[
            pltpu.VMEM((_NBUF, _CHR, _W), jnp.int32),
            pltpu.VMEM((4, _BINS), jnp.float32),
            pltpu.SemaphoreType.DMA((_NBUF,)),
        ],
        compiler_params=pltpu.CompilerParams(needs_layout_passes=False),
    )(functools.partial(_hist_body, rows_per_w))
    return run(t2d)


def kernel(input, target):
    n, c, h, w = input.shape
    t32 = target.astype(jnp.int32)
    n_tot = n * h * w
    hist = _class_hist(t32.reshape(n * h, w))
    sums = _focal_sums(input, t32)
    counts = jnp.sum(hist, axis=0)[:CLS]
    weights = 1.0 / jnp.log(1.1 + counts / n_tot)
    return -jnp.sum(weights * sums[:, 0]) / n_tot


# speedup vs baseline: 1.0070x; 1.0070x over previous
import functools

import jax
import jax.numpy as jnp
from jax import lax
from jax.experimental import pallas as pl
from jax.experimental.pallas import tpu as pltpu
from jax.experimental.pallas import tpu_sc as plsc

CLS = 19
_HB = 256
_SR = 8
_W = 512
_NW = 32
_BINS = 32
_NBUF = 2
_CHR = 16


def _focal_body(nh_blocks, x_ref, t_ref, out_ref, acc_ref):
    step = pl.program_id(0) * pl.num_programs(1) + pl.program_id(1)

    @pl.when(step == 0)
    def _init():
        acc_ref[...] = jnp.zeros_like(acc_ref)

    for r in range(_HB // _SR):
        rows = pl.ds(r * _SR, _SR)
        t = t_ref[0, rows]
        m = x_ref[0, 0, rows]
        for k in range(1, CLS):
            m = jnp.maximum(m, x_ref[0, k, rows])
        s = jnp.zeros((_SR, _W), jnp.float32)
        xt = jnp.zeros((_SR, _W), jnp.float32)
        for k in range(CLS):
            xk = x_ref[0, k, rows]
            s = s + jnp.exp(xk - m)
            xt = xt + jnp.where(t == k, xk, 0.0)
        log_p = xt - m - jnp.log(s)
        p = jnp.exp(log_p)
        om = 1.0 - p
        f = om * om * log_p
        for k in range(CLS):
            acc_ref[k] += jnp.where(t == k, f, 0.0)

    @pl.when(step == nh_blocks - 1)
    def _fin():
        out_ref[...] = jnp.sum(acc_ref[...], axis=(1, 2)).reshape(CLS, 1)


def _focal_sums(x, t32):
    n, c, h, w = x.shape
    grid = (n, h // _HB)
    return pl.pallas_call(
        functools.partial(_focal_body, grid[0] * grid[1]),
        grid=grid,
        in_specs=[
            pl.BlockSpec((1, CLS, _HB, _W), lambda i, j: (i, 0, j, 0)),
            pl.BlockSpec((1, _HB, _W), lambda i, j: (i, j, 0)),
        ],
        out_specs=pl.BlockSpec((CLS, 1), lambda i, j: (0, 0)),
        out_shape=jax.ShapeDtypeStruct((CLS, 1), jnp.float32),
        scratch_shapes=[pltpu.VMEM((CLS, _SR, _W), jnp.float32)],
    )(x, t32)


def _hist_body(rows_per_w, t_hbm, out_hbm, bufs, bins, sems):
    cid = lax.axis_index("c")
    sid = lax.axis_index("s")
    wid = sid * 2 + cid
    row0 = wid * rows_per_w
    nch = rows_per_w // _CHR

    zero = jnp.zeros((16,), jnp.float32)
    for b in range(4):
        for i in range(_BINS // 16):
            bins[b, pl.ds(i * 16, 16)] = zero
    ones = jnp.ones((16,), jnp.float32)

    def start(ci, slot):
        pltpu.async_copy(
            t_hbm.at[pl.ds(row0 + ci * _CHR, _CHR)], bufs.at[slot],
            sems.at[slot])

    for b in range(_NBUF):
        start(b, b)

    def process(ci, carry):
        slot = lax.rem(ci, _NBUF)
        pltpu.make_async_copy(
            t_hbm.at[pl.ds(row0 + ci * _CHR, _CHR)], bufs.at[slot],
            sems.at[slot]).wait()
        for rr in range(_CHR):
            for cc in range(_W // (16 * 4)):
                for u in range(4):
                    tv = bufs[slot, rr, pl.ds(cc * 64 + u * 16, 16)]
                    plsc.addupdate_scatter(bins.at[u], [tv], ones)

        @pl.when(ci + _NBUF < nch)
        def _next():
            start(ci + _NBUF, slot)

        return carry

    lax.fori_loop(0, nch, process, 0)
    for b in range(4):
        pltpu.sync_copy(bins.at[b], out_hbm.at[4 * wid + b])


def _class_hist(t2d):
    rows = t2d.shape[0]
    rows_per_w = rows // _NW
    mesh = plsc.VectorSubcoreMesh(core_axis_name="c", subcore_axis_name="s",
                                  num_cores=2, num_subcores=16)
    run = functools.partial(
        pl.kernel,
        out_type=jax.ShapeDtypeStruct((4 * _NW, _BINS), jnp.float32),
        mesh=mesh,
        scratch_types=[
            pltpu.VMEM((_NBUF, _CHR, _W), jnp.int32),
            pltpu.VMEM((4, _BINS), jnp.float32),
            pltpu.SemaphoreType.DMA((_NBUF,)),
        ],
        compiler_params=pltpu.CompilerParams(needs_layout_passes=False),
    )(functools.partial(_hist_body, rows_per_w))
    return run(t2d)


def kernel(input, target):
    n, c, h, w = input.shape
    t32 = target.astype(jnp.int32)
    n_tot = n * h * w
    hist = _class_hist(t32.reshape(n * h, w))
    sums = _focal_sums(input, t32)
    counts = jnp.sum(hist, axis=0)[:CLS]
    weights = 1.0 / jnp.log(1.1 + counts / n_tot)
    return -jnp.sum(weights * sums[:, 0]) / n_tot
